# baseline (speedup 1.0000x reference)
import functools

import jax
import jax.numpy as jnp
import numpy as np
from jax import lax
from jax.experimental import pallas as pl
from jax.experimental.pallas import tpu as pltpu
from jax.experimental.pallas import tpu_sc as plsc

_P = 0.3
_XMIN = 1.0728769e-07
_ALPHA = 1.0868737
_B, _C, _H, _W = 4, 96, 384, 384

_CHUNK = 96
_NBUF = 3


def _concrete_mask_params():
    key = jax.random.key(42)
    k1, k2, k3, k4 = jax.random.split(key, 4)
    sampled = jax.random.bernoulli(k1, _P, (_B,))
    rand_row = jax.random.randint(k2, (), 0, _H)
    coin = jax.random.bernoulli(k3, 0.5)
    r = jax.random.uniform(k4, (), dtype=jnp.float32)
    rel = jnp.float32(_XMIN) * (1.0 - r) ** (-1.0 / (jnp.float32(_ALPHA) - 1.0))
    return np.asarray(sampled), int(rand_row), bool(coin), np.float32(rel)


_SAMPLED, _RAND_ROW, _COIN, _REL = _concrete_mask_params()


def kernel(forward_input):
    B, C, H, W = forward_input.shape
    R = B * C * H
    x2 = forward_input.reshape(R, W)
    mesh = plsc.VectorSubcoreMesh(core_axis_name="c", subcore_axis_name="s")
    info = plsc.get_sparse_core_info()
    nw = info.num_cores * info.num_subcores
    rows_per = R // nw
    nsteps = rows_per // _CHUNK

    ngroups = nsteps // _NBUF

    @functools.partial(
        pl.kernel,
        mesh=mesh,
        out_type=jax.ShapeDtypeStruct((R, W), jnp.float32),
        scratch_types=[
            pltpu.VMEM((_CHUNK, W), jnp.float32),
            pltpu.VMEM((_CHUNK, W), jnp.float32),
            pltpu.VMEM((_CHUNK, W), jnp.float32),
            pltpu.SemaphoreType.DMA,
            pltpu.SemaphoreType.DMA,
            pltpu.SemaphoreType.DMA,
            pltpu.SemaphoreType.DMA,
            pltpu.SemaphoreType.DMA,
            pltpu.SemaphoreType.DMA,
        ],
    )
    def sc_copy(x_hbm, o_hbm, b0, b1, b2, si0, si1, si2, so0, so1, so2):
        bufs = (b0, b1, b2)
        sin = (si0, si1, si2)
        sout = (so0, so1, so2)
        wid = lax.axis_index("s") * info.num_cores + lax.axis_index("c")
        base = wid * rows_per

        def cin(c, b):
            return pltpu.make_async_copy(
                x_hbm.at[pl.ds(base + c * _CHUNK, _CHUNK)], bufs[b], sin[b]
            )

        def cout(c, b):
            return pltpu.make_async_copy(
                bufs[b], o_hbm.at[pl.ds(base + c * _CHUNK, _CHUNK)], sout[b]
            )

        for b in range(_NBUF):
            cin(b, b).start()

        def body(g, carry):
            for b in range(_NBUF):
                c = g * _NBUF + b
                cin(c, b).wait()
                cout(c, b).start()
            for b in range(_NBUF):
                cout(0, b).wait()
                cin(g * _NBUF + b + _NBUF, b).start()
            return carry

        lax.fori_loop(0, ngroups - 1, body, 0)
        g_last = ngroups - 1
        for b in range(_NBUF):
            cin(g_last * _NBUF + b, b).wait()
            cout(g_last * _NBUF + b, b).start()
        for b in range(_NBUF):
            cout(0, b).wait()

    out = sc_copy(x2)
    return out.reshape(B, C, H, W)


# speedup vs baseline: 1.3178x; 1.3178x over previous
import jax
import jax.numpy as jnp
import numpy as np
from jax.experimental import pallas as pl
from jax.experimental.pallas import tpu as pltpu

_P = 0.3
_XMIN = 1.0728769e-07
_ALPHA = 1.0868737
_B, _C, _H, _W = 4, 96, 384, 384

_BR = 9216


def _concrete_mask_params():
    key = jax.random.key(42)
    k1, k2, k3, k4 = jax.random.split(key, 4)
    sampled = jax.random.bernoulli(k1, _P, (_B,))
    rand_row = jax.random.randint(k2, (), 0, _H)
    coin = jax.random.bernoulli(k3, 0.5)
    r = jax.random.uniform(k4, (), dtype=jnp.float32)
    rel = jnp.float32(_XMIN) * (1.0 - r) ** (-1.0 / (jnp.float32(_ALPHA) - 1.0))
    return (
        np.asarray(sampled),
        int(rand_row),
        bool(coin),
        np.float32(rel),
    )


_SAMPLED, _RAND_ROW, _COIN, _REL = _concrete_mask_params()


def _copy_body(x_ref, o_ref):
    o_ref[...] = x_ref[...]


def _factor_body(rf_ref, cf_ref, x_ref, o_ref):
    o_ref[...] = x_ref[...] * rf_ref[...] * cf_ref[...]


def _factor_call(x3, B, CH, W):
    h = np.arange(CH) % _H
    rf = np.where(
        (not _COIN) & _SAMPLED[:, None] & (h[None, :] == _RAND_ROW),
        _REL,
        np.float32(1.0),
    ).astype(np.float32)[:, :, None]
    cf = np.where(
        _COIN & _SAMPLED[:, None] & (np.arange(W)[None, :] == _RAND_ROW),
        _REL,
        np.float32(1.0),
    ).astype(np.float32)[:, None, :]
    br = 4608
    return pl.pallas_call(
        _factor_body,
        grid=(B, CH // br),
        in_specs=[
            pl.BlockSpec((1, br, 1), lambda b, j: (b, j, 0)),
            pl.BlockSpec((1, 1, W), lambda b, j: (b, 0, 0)),
            pl.BlockSpec((1, br, W), lambda b, j: (b, j, 0)),
        ],
        out_specs=pl.BlockSpec((1, br, W), lambda b, j: (b, j, 0)),
        out_shape=jax.ShapeDtypeStruct((B, CH, W), jnp.float32),
        compiler_params=pltpu.CompilerParams(
            dimension_semantics=("parallel", "arbitrary")
        ),
    )(jnp.asarray(rf), jnp.asarray(cf), x3)


def kernel(forward_input):
    B, C, H, W = forward_input.shape
    R = B * C * H
    x2 = forward_input.reshape(R, W)
    if not _SAMPLED.any():
        out = pl.pallas_call(
            _copy_body,
            grid=(R // _BR,),
            in_specs=[pl.BlockSpec((_BR, W), lambda j: (j, 0))],
            out_specs=pl.BlockSpec((_BR, W), lambda j: (j, 0)),
            out_shape=jax.ShapeDtypeStruct((R, W), jnp.float32),
            compiler_params=pltpu.CompilerParams(
                dimension_semantics=("parallel",)
            ),
        )(x2)
    else:
        out = _factor_call(forward_input.reshape(B, C * H, W), B, C * H, W)
    return out.reshape(B, C, H, W)
